# Initial kernel scaffold; baseline (speedup 1.0000x reference)
import jax
import jax.numpy as jnp
from jax.experimental import pallas as pl


def kernel(input_ids, embed_audio_tokens):
    raise NotImplementedError("write your pallas kernel here")


# speedup vs baseline: 1.0596x; 1.0596x over previous
import functools

import jax
import jax.numpy as jnp
from jax import lax
from jax.experimental import pallas as pl
from jax.experimental.pallas import tpu as pltpu
from jax.experimental.pallas import tpu_sc as plsc

NUM_CODEBOOKS = 32
VOCAB_STRIDE = 2048 + 3
HIDDEN = 1024
BATCH = 2
SEQ = 2048
N_TOKENS = BATCH * SEQ
LANES = 16
H_CHUNKS = HIDDEN // LANES

_info = plsc.get_sparse_core_info()
_NC, _NS = _info.num_cores, _info.num_subcores
NW = _NC * _NS
TOK_PER_W = N_TOKENS // NW
GROUP = 8

_mesh = plsc.VectorSubcoreMesh(core_axis_name="c", subcore_axis_name="s")


@functools.partial(
    pl.kernel,
    mesh=_mesh,
    out_type=jax.ShapeDtypeStruct((N_TOKENS, HIDDEN), jnp.float32),
    scratch_types=[
        pltpu.VMEM((TOK_PER_W, NUM_CODEBOOKS), jnp.int32),
        pltpu.VMEM((NUM_CODEBOOKS, HIDDEN), jnp.float32),
        pltpu.VMEM((GROUP, HIDDEN), jnp.float32),
        pltpu.SemaphoreType.DMA,
    ],
)
def _embed_sum(ids_hbm, table_hbm, out_hbm, idx_v, rows_v, stage_v, gsem):
    wid = lax.axis_index("s") * _NC + lax.axis_index("c")
    base = wid * TOK_PER_W

    pltpu.sync_copy(ids_hbm.at[pl.ds(base, TOK_PER_W)], idx_v)
    offs0 = lax.iota(jnp.int32, LANES) * VOCAB_STRIDE
    offs1 = offs0 + LANES * VOCAB_STRIDE

    def add_offsets(t, carry):
        idx_v[t, pl.ds(0, LANES)] = idx_v[t, pl.ds(0, LANES)] + offs0
        idx_v[t, pl.ds(LANES, LANES)] = idx_v[t, pl.ds(LANES, LANES)] + offs1
        return carry

    lax.fori_loop(0, TOK_PER_W, add_offsets, 0)

    def group_body(g, carry):
        tok0 = g * GROUP
        for j in range(GROUP):
            t = tok0 + j
            pltpu.async_copy(table_hbm.at[idx_v.at[t]], rows_v, gsem).wait()

            def reduce_chunk(c, inner):
                col = pl.ds(c * LANES, LANES)
                acc = rows_v[0, col]
                for r in range(1, NUM_CODEBOOKS):
                    acc = acc + rows_v[r, col]
                stage_v[j, col] = acc
                return inner

            lax.fori_loop(0, H_CHUNKS, reduce_chunk, 0)
        pltpu.sync_copy(stage_v, out_hbm.at[pl.ds(base + tok0, GROUP)])
        return carry

    lax.fori_loop(0, TOK_PER_W // GROUP, group_body, 0)


def kernel(input_ids, embed_audio_tokens):
    ids = input_ids.reshape(N_TOKENS, NUM_CODEBOOKS)
    out = _embed_sum(ids, embed_audio_tokens)
    return out.reshape(BATCH, SEQ, HIDDEN)


# speedup vs baseline: 1.9274x; 1.8189x over previous
import functools

import jax
import jax.numpy as jnp
from jax import lax
from jax.experimental import pallas as pl
from jax.experimental.pallas import tpu as pltpu
from jax.experimental.pallas import tpu_sc as plsc

NUM_CODEBOOKS = 32
VOCAB_STRIDE = 2048 + 3
HIDDEN = 1024
BATCH = 2
SEQ = 2048
N_TOKENS = BATCH * SEQ
LANES = 16
H_CHUNKS = HIDDEN // LANES

_info = plsc.get_sparse_core_info()
_NC, _NS = _info.num_cores, _info.num_subcores
NW = _NC * _NS
TOK_PER_W = N_TOKENS // NW
GROUP = 8

_mesh = plsc.VectorSubcoreMesh(core_axis_name="c", subcore_axis_name="s")


@functools.partial(
    pl.kernel,
    mesh=_mesh,
    out_type=jax.ShapeDtypeStruct((N_TOKENS, HIDDEN), jnp.float32),
    scratch_types=[
        pltpu.VMEM((TOK_PER_W, NUM_CODEBOOKS), jnp.int32),
        pltpu.VMEM((2, NUM_CODEBOOKS, HIDDEN), jnp.float32),
        pltpu.VMEM((GROUP, HIDDEN), jnp.float32),
        pltpu.SemaphoreType.DMA,
        pltpu.SemaphoreType.DMA,
    ],
)
def _embed_sum(ids_hbm, table_hbm, out_hbm, idx_v, rows_v, stage_v, gsem0, gsem1):
    wid = lax.axis_index("s") * _NC + lax.axis_index("c")
    base = wid * TOK_PER_W

    pltpu.sync_copy(ids_hbm.at[pl.ds(base, TOK_PER_W)], idx_v)
    offs0 = lax.iota(jnp.int32, LANES) * VOCAB_STRIDE
    offs1 = offs0 + LANES * VOCAB_STRIDE

    def add_offsets(t, carry):
        idx_v[t, pl.ds(0, LANES)] = idx_v[t, pl.ds(0, LANES)] + offs0
        idx_v[t, pl.ds(LANES, LANES)] = idx_v[t, pl.ds(LANES, LANES)] + offs1
        return carry

    lax.fori_loop(0, TOK_PER_W, add_offsets, 0)

    sems = (gsem0, gsem1)

    def gather(t, slot):
        return pltpu.make_async_copy(
            table_hbm.at[idx_v.at[t]], rows_v.at[slot], sems[slot])

    gather(0, 0).start()

    def group_body(g, carry):
        tok0 = g * GROUP
        for j in range(GROUP):
            t = tok0 + j
            slot = j % 2
            nxt = (j + 1) % 2

            @pl.when(t + 1 < TOK_PER_W)
            def _():
                gather(t + 1, nxt).start()

            gather(t, slot).wait()

            def reduce_chunk(c, inner):
                col = pl.ds(c * LANES, LANES)
                acc = rows_v[slot, 0, col]
                for r in range(1, NUM_CODEBOOKS):
                    acc = acc + rows_v[slot, r, col]
                stage_v[j, col] = acc
                return inner

            lax.fori_loop(0, H_CHUNKS, reduce_chunk, 0)
        pltpu.sync_copy(stage_v, out_hbm.at[pl.ds(base + tok0, GROUP)])
        return carry

    lax.fori_loop(0, TOK_PER_W // GROUP, group_body, 0)


def kernel(input_ids, embed_audio_tokens):
    ids = input_ids.reshape(N_TOKENS, NUM_CODEBOOKS)
    out = _embed_sum(ids, embed_audio_tokens)
    return out.reshape(BATCH, SEQ, HIDDEN)


# speedup vs baseline: 2.0989x; 1.0890x over previous
import functools

import jax
import jax.numpy as jnp
from jax import lax
from jax.experimental import pallas as pl
from jax.experimental.pallas import tpu as pltpu
from jax.experimental.pallas import tpu_sc as plsc

NUM_CODEBOOKS = 32
VOCAB_STRIDE = 2048 + 3
HIDDEN = 1024
BATCH = 2
SEQ = 2048
N_TOKENS = BATCH * SEQ
LANES = 16
H_CHUNKS = HIDDEN // LANES

_info = plsc.get_sparse_core_info()
_NC, _NS = _info.num_cores, _info.num_subcores
NW = _NC * _NS
TOK_PER_W = N_TOKENS // NW
GROUP = 8

_mesh = plsc.VectorSubcoreMesh(core_axis_name="c", subcore_axis_name="s")


@functools.partial(
    pl.kernel,
    mesh=_mesh,
    out_type=jax.ShapeDtypeStruct((N_TOKENS, HIDDEN), jnp.float32),
    scratch_types=[
        pltpu.VMEM((TOK_PER_W, NUM_CODEBOOKS), jnp.int32),
        pltpu.VMEM((2, NUM_CODEBOOKS, HIDDEN), jnp.float32),
        pltpu.VMEM((GROUP, HIDDEN), jnp.float32),
        pltpu.SemaphoreType.DMA,
        pltpu.SemaphoreType.DMA,
    ],
)
def _embed_sum(ids_hbm, table_hbm, out_hbm, idx_v, rows_v, stage_v, gsem0, gsem1):
    wid = lax.axis_index("s") * _NC + lax.axis_index("c")
    base = wid * TOK_PER_W

    pltpu.sync_copy(ids_hbm.at[pl.ds(base, TOK_PER_W)], idx_v)
    offs0 = lax.iota(jnp.int32, LANES) * VOCAB_STRIDE
    offs1 = offs0 + LANES * VOCAB_STRIDE

    def add_offsets(t, carry):
        idx_v[t, pl.ds(0, LANES)] = idx_v[t, pl.ds(0, LANES)] + offs0
        idx_v[t, pl.ds(LANES, LANES)] = idx_v[t, pl.ds(LANES, LANES)] + offs1
        return carry

    lax.fori_loop(0, TOK_PER_W, add_offsets, 0)

    sems = (gsem0, gsem1)

    def gather(t, slot):
        return pltpu.make_async_copy(
            table_hbm.at[idx_v.at[t]], rows_v.at[slot], sems[slot])

    gather(0, 0).start()

    def group_body(g, carry):
        tok0 = g * GROUP
        for j in range(GROUP):
            t = tok0 + j
            slot = j % 2
            nxt = (j + 1) % 2

            @pl.when(t + 1 < TOK_PER_W)
            def _():
                gather(t + 1, nxt).start()

            gather(t, slot).wait()

            def reduce_chunk(c, inner):
                for u in range(2):
                    col = pl.ds((c * 2 + u) * LANES, LANES)
                    vals = [rows_v[slot, r, col] for r in range(NUM_CODEBOOKS)]
                    while len(vals) > 1:
                        vals = [vals[i] + vals[i + 1]
                                for i in range(0, len(vals), 2)]
                    stage_v[j, col] = vals[0]
                return inner

            lax.fori_loop(0, H_CHUNKS // 2, reduce_chunk, 0)
        pltpu.sync_copy(stage_v, out_hbm.at[pl.ds(base + tok0, GROUP)])
        return carry

    lax.fori_loop(0, TOK_PER_W // GROUP, group_body, 0)


def kernel(input_ids, embed_audio_tokens):
    ids = input_ids.reshape(N_TOKENS, NUM_CODEBOOKS)
    out = _embed_sum(ids, embed_audio_tokens)
    return out.reshape(BATCH, SEQ, HIDDEN)


# speedup vs baseline: 2.1034x; 1.0022x over previous
import functools

import jax
import jax.numpy as jnp
from jax import lax
from jax.experimental import pallas as pl
from jax.experimental.pallas import tpu as pltpu
from jax.experimental.pallas import tpu_sc as plsc

NUM_CODEBOOKS = 32
VOCAB_STRIDE = 2048 + 3
HIDDEN = 1024
BATCH = 2
SEQ = 2048
N_TOKENS = BATCH * SEQ
LANES = 16
H_CHUNKS = HIDDEN // LANES

_info = plsc.get_sparse_core_info()
_NC, _NS = _info.num_cores, _info.num_subcores
NW = _NC * _NS
TOK_PER_W = N_TOKENS // NW
GROUP = 8

_mesh = plsc.VectorSubcoreMesh(core_axis_name="c", subcore_axis_name="s")


@functools.partial(
    pl.kernel,
    mesh=_mesh,
    out_type=jax.ShapeDtypeStruct((N_TOKENS, HIDDEN), jnp.float32),
    scratch_types=[
        pltpu.VMEM((TOK_PER_W, NUM_CODEBOOKS), jnp.int32),
        pltpu.VMEM((2, NUM_CODEBOOKS, HIDDEN), jnp.float32),
        pltpu.VMEM((GROUP, HIDDEN), jnp.float32),
        pltpu.SemaphoreType.DMA,
        pltpu.SemaphoreType.DMA,
        pltpu.SemaphoreType.DMA,
        pltpu.SemaphoreType.DMA,
    ],
)
def _embed_sum(ids_hbm, table_hbm, out_hbm, idx_v, rows_v, stage_v,
               gsem00, gsem01, gsem10, gsem11):
    wid = lax.axis_index("s") * _NC + lax.axis_index("c")
    base = wid * TOK_PER_W

    pltpu.sync_copy(ids_hbm.at[pl.ds(base, TOK_PER_W)], idx_v)
    offs0 = lax.iota(jnp.int32, LANES) * VOCAB_STRIDE
    offs1 = offs0 + LANES * VOCAB_STRIDE

    def add_offsets(t, carry):
        idx_v[t, pl.ds(0, LANES)] = idx_v[t, pl.ds(0, LANES)] + offs0
        idx_v[t, pl.ds(LANES, LANES)] = idx_v[t, pl.ds(LANES, LANES)] + offs1
        return carry

    lax.fori_loop(0, TOK_PER_W, add_offsets, 0)

    sems = ((gsem00, gsem01), (gsem10, gsem11))
    HALF = NUM_CODEBOOKS // 2

    def gather(t, slot, h):
        return pltpu.make_async_copy(
            table_hbm.at[idx_v.at[t, pl.ds(h * HALF, HALF)]],
            rows_v.at[slot, pl.ds(h * HALF, HALF)],
            sems[slot][h])

    gather(0, 0, 0).start()
    gather(0, 0, 1).start()

    def group_body(g, carry):
        tok0 = g * GROUP
        for j in range(GROUP):
            t = tok0 + j
            slot = j % 2
            nxt = (j + 1) % 2

            @pl.when(t + 1 < TOK_PER_W)
            def _():
                gather(t + 1, nxt, 0).start()
                gather(t + 1, nxt, 1).start()

            gather(t, slot, 0).wait()
            gather(t, slot, 1).wait()

            def reduce_chunk(c, inner):
                for u in range(2):
                    col = pl.ds((c * 2 + u) * LANES, LANES)
                    vals = [rows_v[slot, r, col] for r in range(NUM_CODEBOOKS)]
                    while len(vals) > 1:
                        vals = [vals[i] + vals[i + 1]
                                for i in range(0, len(vals), 2)]
                    stage_v[j, col] = vals[0]
                return inner

            lax.fori_loop(0, H_CHUNKS // 2, reduce_chunk, 0)
        pltpu.sync_copy(stage_v, out_hbm.at[pl.ds(base + tok0, GROUP)])
        return carry

    lax.fori_loop(0, TOK_PER_W // GROUP, group_body, 0)


def kernel(input_ids, embed_audio_tokens):
    ids = input_ids.reshape(N_TOKENS, NUM_CODEBOOKS)
    out = _embed_sum(ids, embed_audio_tokens)
    return out.reshape(BATCH, SEQ, HIDDEN)
